# baseline (speedup 1.0000x reference)
import functools

import jax
import jax.numpy as jnp
from jax import lax
from jax.experimental import pallas as pl
from jax.experimental.pallas import tpu as pltpu
from jax.experimental.pallas import tpu_sc as plsc

EMBED_DIM = 256
NUM_CODES = 1024
BN = 16384
HALF = BN // 2
BLK = 512

_DN_T = (((1,), (1,)), ((), ()))

_info = plsc.get_sparse_core_info()
_NC, _NS = _info.num_cores, _info.num_subcores
_NW = _NC * _NS
_CHUNK = 256


def _make_sc_gather(n):
    b_per_w = n // _NW

    def body(table_hbm, idx_hbm, out_hbm, idx_v, rows_v, sem):
        wid = lax.axis_index("s") * _NC + lax.axis_index("c")
        base = wid * b_per_w
        pltpu.sync_copy(idx_hbm.at[pl.ds(base, b_per_w)], idx_v)
        for ch in range(max(1, b_per_w // _CHUNK)):
            rows = min(_CHUNK, b_per_w)
            pltpu.async_copy(
                table_hbm.at[idx_v.at[pl.ds(ch * rows, rows)]], rows_v, sem
            ).wait()
            pltpu.sync_copy(rows_v,
                            out_hbm.at[pl.ds(base + ch * rows, rows)])
    return body, b_per_w


def _gather_rows(table, idx_flat, n):
    body, b_per_w = _make_sc_gather(n)
    mesh = plsc.VectorSubcoreMesh(core_axis_name="c", subcore_axis_name="s")
    k = functools.partial(
        pl.kernel, mesh=mesh,
        out_type=jax.ShapeDtypeStruct((n, EMBED_DIM), jnp.float32),
        scratch_types=[pltpu.VMEM((b_per_w,), jnp.int32),
                       pltpu.VMEM((min(_CHUNK, b_per_w), EMBED_DIM),
                                  jnp.float32),
                       pltpu.SemaphoreType.DMA],
    )(body)
    return k(table, idx_flat)


def _dist_argmin(r, cb):
    cbsq = jnp.sum(cb * cb, axis=1)
    rsq = jnp.sum(r * r, axis=1, keepdims=True)
    prod = jax.lax.dot_general(r, cb, _DN_T,
                               preferred_element_type=jnp.float32)
    dist = rsq - 2.0 * prod + cbsq[None, :]
    mind = jnp.min(dist, axis=1, keepdims=True)
    lane = jax.lax.broadcasted_iota(
        jnp.int32, (1, NUM_CODES), 1).astype(jnp.float32)
    idxf = jnp.min(jnp.where(dist == mind, lane, jnp.float32(NUM_CODES)),
                   axis=1)
    return idxf.astype(jnp.int32)


def _level0_body(r_ref, cb_ref, idx_ref):
    idx_ref[0, 0, :] = _dist_argmin(r_ref[...], cb_ref[...])


def _levelN_body(rp_ref, qp_ref, cb_ref, idx_ref, rn_ref, ls_ref):
    @pl.when(pl.program_id(0) == 0)
    def _():
        ls_ref[0, 0] = jnp.float32(0.0)
    r = rp_ref[...] - qp_ref[...]
    rn_ref[...] = r
    idx_ref[0, 0, :] = _dist_argmin(r, cb_ref[...])
    ls_ref[0, 0] += jnp.sum(r * r)


def _final_body(z_ref, rp_ref, qp_ref, zq_ref, ls_ref):
    @pl.when(pl.program_id(0) == 0)
    def _():
        ls_ref[0, 0] = jnp.float32(0.0)
    r = rp_ref[...] - qp_ref[...]
    zq_ref[...] = z_ref[...] - r
    ls_ref[0, 0] += jnp.sum(r * r)


_rspec = pl.BlockSpec((BLK, EMBED_DIM), lambda i: (i, 0))
_cspec = pl.BlockSpec((NUM_CODES, EMBED_DIM), lambda i: (0, 0))
_ispec = pl.BlockSpec((1, 1, BLK), lambda i: (i, 0, 0))
_sspec = pl.BlockSpec(memory_space=pltpu.SMEM)
_sshape = jax.ShapeDtypeStruct((1, 1), jnp.float32)
_arb = pltpu.CompilerParams(dimension_semantics=("arbitrary",))


def _half_pipeline(flat, codebook0, codebook1, codebook2):
    n = flat.shape[0]
    nb = n // BLK
    ishape = jax.ShapeDtypeStruct((nb, 1, BLK), jnp.int32)
    fshape = jax.ShapeDtypeStruct((n, EMBED_DIM), jnp.float32)

    i0 = pl.pallas_call(_level0_body, grid=(nb,),
                        in_specs=[_rspec, _cspec], out_specs=_ispec,
                        out_shape=ishape, compiler_params=_arb)(flat, codebook0)
    q0 = _gather_rows(codebook0, i0.reshape(n), n)

    i1, r1, s0 = pl.pallas_call(_levelN_body, grid=(nb,),
                                in_specs=[_rspec, _rspec, _cspec],
                                out_specs=[_ispec, _rspec, _sspec],
                                out_shape=[ishape, fshape, _sshape],
                                compiler_params=_arb)(flat, q0, codebook1)
    q1 = _gather_rows(codebook1, i1.reshape(n), n)

    i2, r2, s1 = pl.pallas_call(_levelN_body, grid=(nb,),
                                in_specs=[_rspec, _rspec, _cspec],
                                out_specs=[_ispec, _rspec, _sspec],
                                out_shape=[ishape, fshape, _sshape],
                                compiler_params=_arb)(r1, q1, codebook2)
    q2 = _gather_rows(codebook2, i2.reshape(n), n)

    zq, s2 = pl.pallas_call(_final_body, grid=(nb,),
                            in_specs=[_rspec, _rspec, _rspec],
                            out_specs=[_rspec, _sspec],
                            out_shape=[fshape, _sshape],
                            compiler_params=_arb)(flat, r2, q2)

    return zq, (i0.reshape(n), i1.reshape(n), i2.reshape(n)), (s0, s1, s2)


@jax.jit
def kernel(z_e, codebook0, codebook1, codebook2):
    B, N, D = z_e.shape
    flat = z_e.reshape(BN, D)

    zq, idx, s = _half_pipeline(flat, codebook0, codebook1, codebook2)

    scale = jnp.float32(1.0 / (BN * EMBED_DIM))
    loss = (1.25 * s[0][0, 0] + 1.5 * s[1][0, 0]
            + 2.0 * s[2][0, 0]) * scale
    return (zq.reshape(B, N, D), loss, idx[0].reshape(B, N),
            idx[1].reshape(B, N), idx[2].reshape(B, N))


# speedup vs baseline: 1.1000x; 1.1000x over previous
import functools

import jax
import jax.numpy as jnp
from jax import lax
from jax.experimental import pallas as pl
from jax.experimental.pallas import tpu as pltpu
from jax.experimental.pallas import tpu_sc as plsc

EMBED_DIM = 256
NUM_CODES = 1024
BN = 16384
HALF = BN // 2
BLK = 2048

_DN_T = (((1,), (1,)), ((), ()))

_info = plsc.get_sparse_core_info()
_NC, _NS = _info.num_cores, _info.num_subcores
_NW = _NC * _NS
_CHUNK = 256


def _make_sc_gather(n):
    b_per_w = n // _NW

    def body(table_hbm, idx_hbm, out_hbm, idx_v, rows_v, sem):
        wid = lax.axis_index("s") * _NC + lax.axis_index("c")
        base = wid * b_per_w
        pltpu.sync_copy(idx_hbm.at[pl.ds(base, b_per_w)], idx_v)
        for ch in range(max(1, b_per_w // _CHUNK)):
            rows = min(_CHUNK, b_per_w)
            pltpu.async_copy(
                table_hbm.at[idx_v.at[pl.ds(ch * rows, rows)]], rows_v, sem
            ).wait()
            pltpu.sync_copy(rows_v,
                            out_hbm.at[pl.ds(base + ch * rows, rows)])
    return body, b_per_w


def _gather_rows(table, idx_flat, n):
    body, b_per_w = _make_sc_gather(n)
    mesh = plsc.VectorSubcoreMesh(core_axis_name="c", subcore_axis_name="s")
    k = functools.partial(
        pl.kernel, mesh=mesh,
        out_type=jax.ShapeDtypeStruct((n, EMBED_DIM), jnp.float32),
        scratch_types=[pltpu.VMEM((b_per_w,), jnp.int32),
                       pltpu.VMEM((min(_CHUNK, b_per_w), EMBED_DIM),
                                  jnp.float32),
                       pltpu.SemaphoreType.DMA],
    )(body)
    return k(table, idx_flat)


def _dist_argmin(r, cb):
    cbsq = jnp.sum(cb * cb, axis=1)
    rsq = jnp.sum(r * r, axis=1, keepdims=True)
    prod = jax.lax.dot_general(r, cb, _DN_T,
                               preferred_element_type=jnp.float32)
    dist = rsq - 2.0 * prod + cbsq[None, :]
    mind = jnp.min(dist, axis=1, keepdims=True)
    lane = jax.lax.broadcasted_iota(
        jnp.int32, (1, NUM_CODES), 1).astype(jnp.float32)
    idxf = jnp.min(jnp.where(dist == mind, lane, jnp.float32(NUM_CODES)),
                   axis=1)
    return idxf.astype(jnp.int32)


def _level0_body(r_ref, cb_ref, idx_ref):
    idx_ref[0, 0, :] = _dist_argmin(r_ref[...], cb_ref[...])


def _levelN_body(rp_ref, qp_ref, cb_ref, idx_ref, rn_ref, ls_ref):
    @pl.when(pl.program_id(0) == 0)
    def _():
        ls_ref[0, 0] = jnp.float32(0.0)
    r = rp_ref[...] - qp_ref[...]
    rn_ref[...] = r
    idx_ref[0, 0, :] = _dist_argmin(r, cb_ref[...])
    ls_ref[0, 0] += jnp.sum(r * r)


def _final_body(z_ref, rp_ref, qp_ref, zq_ref, ls_ref):
    @pl.when(pl.program_id(0) == 0)
    def _():
        ls_ref[0, 0] = jnp.float32(0.0)
    r = rp_ref[...] - qp_ref[...]
    zq_ref[...] = z_ref[...] - r
    ls_ref[0, 0] += jnp.sum(r * r)


_rspec = pl.BlockSpec((BLK, EMBED_DIM), lambda i: (i, 0))
_cspec = pl.BlockSpec((NUM_CODES, EMBED_DIM), lambda i: (0, 0))
_ispec = pl.BlockSpec((1, 1, BLK), lambda i: (i, 0, 0))
_sspec = pl.BlockSpec(memory_space=pltpu.SMEM)
_sshape = jax.ShapeDtypeStruct((1, 1), jnp.float32)
_arb = pltpu.CompilerParams(dimension_semantics=("arbitrary",))


def _half_pipeline(flat, codebook0, codebook1, codebook2):
    n = flat.shape[0]
    nb = n // BLK
    ishape = jax.ShapeDtypeStruct((nb, 1, BLK), jnp.int32)
    fshape = jax.ShapeDtypeStruct((n, EMBED_DIM), jnp.float32)

    i0 = pl.pallas_call(_level0_body, grid=(nb,),
                        in_specs=[_rspec, _cspec], out_specs=_ispec,
                        out_shape=ishape, compiler_params=_arb)(flat, codebook0)
    q0 = _gather_rows(codebook0, i0.reshape(n), n)

    i1, r1, s0 = pl.pallas_call(_levelN_body, grid=(nb,),
                                in_specs=[_rspec, _rspec, _cspec],
                                out_specs=[_ispec, _rspec, _sspec],
                                out_shape=[ishape, fshape, _sshape],
                                compiler_params=_arb)(flat, q0, codebook1)
    q1 = _gather_rows(codebook1, i1.reshape(n), n)

    i2, r2, s1 = pl.pallas_call(_levelN_body, grid=(nb,),
                                in_specs=[_rspec, _rspec, _cspec],
                                out_specs=[_ispec, _rspec, _sspec],
                                out_shape=[ishape, fshape, _sshape],
                                compiler_params=_arb)(r1, q1, codebook2)
    q2 = _gather_rows(codebook2, i2.reshape(n), n)

    zq, s2 = pl.pallas_call(_final_body, grid=(nb,),
                            in_specs=[_rspec, _rspec, _rspec],
                            out_specs=[_rspec, _sspec],
                            out_shape=[fshape, _sshape],
                            compiler_params=_arb)(flat, r2, q2)

    return zq, (i0.reshape(n), i1.reshape(n), i2.reshape(n)), (s0, s1, s2)


@jax.jit
def kernel(z_e, codebook0, codebook1, codebook2):
    B, N, D = z_e.shape
    flat = z_e.reshape(BN, D)

    zq, idx, s = _half_pipeline(flat, codebook0, codebook1, codebook2)

    scale = jnp.float32(1.0 / (BN * EMBED_DIM))
    loss = (1.25 * s[0][0, 0] + 1.5 * s[1][0, 0]
            + 2.0 * s[2][0, 0]) * scale
    return (zq.reshape(B, N, D), loss, idx[0].reshape(B, N),
            idx[1].reshape(B, N), idx[2].reshape(B, N))


# speedup vs baseline: 1.1138x; 1.0126x over previous
import functools

import jax
import jax.numpy as jnp
from jax import lax
from jax.experimental import pallas as pl
from jax.experimental.pallas import tpu as pltpu
from jax.experimental.pallas import tpu_sc as plsc

EMBED_DIM = 256
NUM_CODES = 1024
BN = 16384
HALF = BN // 2
BLK = 4096

_DN_T = (((1,), (1,)), ((), ()))

_info = plsc.get_sparse_core_info()
_NC, _NS = _info.num_cores, _info.num_subcores
_NW = _NC * _NS
_CHUNK = 256


def _make_sc_gather(n):
    b_per_w = n // _NW

    def body(table_hbm, idx_hbm, out_hbm, idx_v, rows_v, sem):
        wid = lax.axis_index("s") * _NC + lax.axis_index("c")
        base = wid * b_per_w
        pltpu.sync_copy(idx_hbm.at[pl.ds(base, b_per_w)], idx_v)
        for ch in range(max(1, b_per_w // _CHUNK)):
            rows = min(_CHUNK, b_per_w)
            pltpu.async_copy(
                table_hbm.at[idx_v.at[pl.ds(ch * rows, rows)]], rows_v, sem
            ).wait()
            pltpu.sync_copy(rows_v,
                            out_hbm.at[pl.ds(base + ch * rows, rows)])
    return body, b_per_w


def _gather_rows(table, idx_flat, n):
    body, b_per_w = _make_sc_gather(n)
    mesh = plsc.VectorSubcoreMesh(core_axis_name="c", subcore_axis_name="s")
    k = functools.partial(
        pl.kernel, mesh=mesh,
        out_type=jax.ShapeDtypeStruct((n, EMBED_DIM), jnp.float32),
        scratch_types=[pltpu.VMEM((b_per_w,), jnp.int32),
                       pltpu.VMEM((min(_CHUNK, b_per_w), EMBED_DIM),
                                  jnp.float32),
                       pltpu.SemaphoreType.DMA],
    )(body)
    return k(table, idx_flat)


def _dist_argmin(r, cb):
    cbsq = jnp.sum(cb * cb, axis=1)
    rsq = jnp.sum(r * r, axis=1, keepdims=True)
    prod = jax.lax.dot_general(r, cb, _DN_T,
                               preferred_element_type=jnp.float32)
    dist = rsq - 2.0 * prod + cbsq[None, :]
    mind = jnp.min(dist, axis=1, keepdims=True)
    lane = jax.lax.broadcasted_iota(
        jnp.int32, (1, NUM_CODES), 1).astype(jnp.float32)
    idxf = jnp.min(jnp.where(dist == mind, lane, jnp.float32(NUM_CODES)),
                   axis=1)
    return idxf.astype(jnp.int32)


def _level0_body(r_ref, cb_ref, idx_ref):
    idx_ref[0, 0, :] = _dist_argmin(r_ref[...], cb_ref[...])


def _levelN_body(rp_ref, qp_ref, cb_ref, idx_ref, rn_ref, ls_ref):
    @pl.when(pl.program_id(0) == 0)
    def _():
        ls_ref[0, 0] = jnp.float32(0.0)
    r = rp_ref[...] - qp_ref[...]
    rn_ref[...] = r
    idx_ref[0, 0, :] = _dist_argmin(r, cb_ref[...])
    ls_ref[0, 0] += jnp.sum(r * r)


def _final_body(z_ref, rp_ref, qp_ref, zq_ref, ls_ref):
    @pl.when(pl.program_id(0) == 0)
    def _():
        ls_ref[0, 0] = jnp.float32(0.0)
    r = rp_ref[...] - qp_ref[...]
    zq_ref[...] = z_ref[...] - r
    ls_ref[0, 0] += jnp.sum(r * r)


_rspec = pl.BlockSpec((BLK, EMBED_DIM), lambda i: (i, 0))
_cspec = pl.BlockSpec((NUM_CODES, EMBED_DIM), lambda i: (0, 0))
_ispec = pl.BlockSpec((1, 1, BLK), lambda i: (i, 0, 0))
_sspec = pl.BlockSpec(memory_space=pltpu.SMEM)
_sshape = jax.ShapeDtypeStruct((1, 1), jnp.float32)
_arb = pltpu.CompilerParams(dimension_semantics=("arbitrary",))


def _half_pipeline(flat, codebook0, codebook1, codebook2):
    n = flat.shape[0]
    nb = n // BLK
    ishape = jax.ShapeDtypeStruct((nb, 1, BLK), jnp.int32)
    fshape = jax.ShapeDtypeStruct((n, EMBED_DIM), jnp.float32)

    i0 = pl.pallas_call(_level0_body, grid=(nb,),
                        in_specs=[_rspec, _cspec], out_specs=_ispec,
                        out_shape=ishape, compiler_params=_arb)(flat, codebook0)
    q0 = _gather_rows(codebook0, i0.reshape(n), n)

    i1, r1, s0 = pl.pallas_call(_levelN_body, grid=(nb,),
                                in_specs=[_rspec, _rspec, _cspec],
                                out_specs=[_ispec, _rspec, _sspec],
                                out_shape=[ishape, fshape, _sshape],
                                compiler_params=_arb)(flat, q0, codebook1)
    q1 = _gather_rows(codebook1, i1.reshape(n), n)

    i2, r2, s1 = pl.pallas_call(_levelN_body, grid=(nb,),
                                in_specs=[_rspec, _rspec, _cspec],
                                out_specs=[_ispec, _rspec, _sspec],
                                out_shape=[ishape, fshape, _sshape],
                                compiler_params=_arb)(r1, q1, codebook2)
    q2 = _gather_rows(codebook2, i2.reshape(n), n)

    zq, s2 = pl.pallas_call(_final_body, grid=(nb,),
                            in_specs=[_rspec, _rspec, _rspec],
                            out_specs=[_rspec, _sspec],
                            out_shape=[fshape, _sshape],
                            compiler_params=_arb)(flat, r2, q2)

    return zq, (i0.reshape(n), i1.reshape(n), i2.reshape(n)), (s0, s1, s2)


@jax.jit
def kernel(z_e, codebook0, codebook1, codebook2):
    B, N, D = z_e.shape
    flat = z_e.reshape(BN, D)

    zq, idx, s = _half_pipeline(flat, codebook0, codebook1, codebook2)

    scale = jnp.float32(1.0 / (BN * EMBED_DIM))
    loss = (1.25 * s[0][0, 0] + 1.5 * s[1][0, 0]
            + 2.0 * s[2][0, 0]) * scale
    return (zq.reshape(B, N, D), loss, idx[0].reshape(B, N),
            idx[1].reshape(B, N), idx[2].reshape(B, N))
